# baseline (speedup 1.0000x reference)
import dataclasses
import functools

import jax
import jax.numpy as jnp
from jax import lax
from jax.experimental import pallas as pl
from jax.experimental.pallas import tpu as pltpu
from jax.experimental.pallas import tpu_sc as plsc

N = 10000
NPAD = 10240
E = 320000
F = 128

NCORES = 2
NSUB = 16
NTILES = NCORES * NSUB
EPT = E // NTILES
CH = 40
NCH = EPT // CH
CPB = 50
NBLK = NCH // CPB
NBUF = 4
PD = 2
ZR = 128
RPT = NPAD // NSUB

_mesh = plsc.VectorSubcoreMesh(core_axis_name="c", subcore_axis_name="s")

_sc_params = pltpu.CompilerParams()
if "needs_layout_passes" in pltpu.CompilerParams.__dataclass_fields__:
    _sc_params = dataclasses.replace(_sc_params, needs_layout_passes=False)


def _spmm_hop(table, col3, row3, w3):
    @functools.partial(
        pl.kernel,
        out_type=jax.ShapeDtypeStruct((NCORES, NPAD, F), jnp.float32),
        mesh=_mesh,
        compiler_params=_sc_params,
        scratch_types=[
            pltpu.VMEM((CPB, CH), jnp.int32),
            pltpu.VMEM((CPB, CH), jnp.int32),
            pltpu.VMEM((CPB, CH), jnp.float32),
            pltpu.VMEM((CH, F), jnp.float32),
            pltpu.VMEM((CH, F), jnp.float32),
            pltpu.VMEM((CH, F), jnp.float32),
            pltpu.VMEM((CH, F), jnp.float32),
            pltpu.VMEM_SHARED((NPAD, F), jnp.float32),
            pltpu.SemaphoreType.DMA,
            pltpu.SemaphoreType.DMA,
            pltpu.SemaphoreType.DMA,
            pltpu.SemaphoreType.DMA,
            pltpu.SemaphoreType.DMA,
            pltpu.SemaphoreType.DMA,
            pltpu.SemaphoreType.DMA,
            pltpu.SemaphoreType.DMA,
        ],
    )
    def hop(table_hbm, col_hbm, row_hbm, w_hbm, out_hbm,
            col_t, row_t, w_t, rows0, rows1, rows2, rows3, acc,
            g0, g1, g2, g3, s0, s1, s2, s3):
        c = lax.axis_index("c")
        s = lax.axis_index("s")
        t = c * NSUB + s
        rows = (rows0, rows1, rows2, rows3)
        gsem = (g0, g1, g2, g3)
        ssem = (s0, s1, s2, s3)

        @pl.loop(0, RPT // ZR)
        def _(i):
            pltpu.sync_copy(table_hbm.at[pl.ds(N, ZR)],
                            acc.at[pl.ds(s * RPT + i * ZR, ZR)])

        plsc.subcore_barrier()

        def scale(kk, b):
            @pl.loop(0, CH)
            def _(e):
                wv = plsc.load_gather(
                    w_t,
                    [jnp.full((16,), kk, jnp.int32), jnp.full((16,), e, jnp.int32)],
                )
                for j in range(F // 16):
                    sl = (e, pl.ds(j * 16, 16))
                    rows[b][sl] = rows[b][sl] * wv

        def gather(kk, b):
            pltpu.async_copy(table_hbm.at[col_t.at[kk]], rows[b], gsem[b])

        def wait_gather(kk, b):
            pltpu.make_async_copy(
                table_hbm.at[col_t.at[kk]], rows[b], gsem[b]).wait()

        def scatter(kk, b):
            pltpu.async_copy(rows[b], acc.at[row_t.at[kk]], ssem[b], add=True)

        def wait_scatter(kk, b):
            pltpu.make_async_copy(rows[b], acc.at[row_t.at[kk]], ssem[b]).wait()

        @pl.loop(0, NBLK)
        def _(blk):
            pltpu.sync_copy(col_hbm.at[t].at[blk], col_t)
            pltpu.sync_copy(row_hbm.at[t].at[blk], row_t)
            pltpu.sync_copy(w_hbm.at[t].at[blk], w_t)

            gather(0, 0)
            gather(1, 1)
            for kk in (0, 1):
                b = kk % NBUF
                wait_gather(kk, b)
                scale(kk, b)
                scatter(kk, b)
                gather(kk + PD, (kk + PD) % NBUF)

            @pl.loop(PD, CPB - NBUF, step=NBUF)
            def _(k):
                for j in range(NBUF):
                    kk = k + j
                    b = (PD + j) % NBUF
                    wait_gather(kk, b)
                    scale(kk, b)
                    scatter(kk, b)
                    b2 = (b + PD) % NBUF
                    wait_scatter(kk - PD, b2)
                    gather(kk + PD, b2)

            for kk in (CPB - 4, CPB - 3):
                b = kk % NBUF
                wait_gather(kk, b)
                scale(kk, b)
                scatter(kk, b)
                b2 = (b + PD) % NBUF
                wait_scatter(kk - PD, b2)
                gather(kk + PD, b2)
            for kk in (CPB - 2, CPB - 1):
                b = kk % NBUF
                wait_gather(kk, b)
                scale(kk, b)
                scatter(kk, b)
            for kk in range(CPB - NBUF, CPB):
                wait_scatter(kk, kk % NBUF)

        plsc.subcore_barrier()

        @pl.loop(0, RPT // ZR)
        def _(i):
            off = s * RPT + i * ZR
            pltpu.sync_copy(acc.at[pl.ds(off, ZR)], out_hbm.at[c].at[pl.ds(off, ZR)])

    return hop(table, col3, row3, w3)


def _combine(p):
    def body(p_ref, o_ref):
        o_ref[...] = p_ref[0] + p_ref[1]

    return pl.pallas_call(
        body,
        out_shape=jax.ShapeDtypeStruct((NPAD, F), jnp.float32),
        grid=(8,),
        in_specs=[pl.BlockSpec((NCORES, NPAD // 8, F), lambda i: (0, i, 0))],
        out_specs=pl.BlockSpec((NPAD // 8, F), lambda i: (i, 0)),
    )(p)


def _final(p, W, b2):
    def body(p_ref, w_ref, b_ref, o_ref):
        h = p_ref[0] + p_ref[1]
        o_ref[...] = lax.dot_general(
            h, w_ref[...], (((1,), (1,)), ((), ())),
            precision=lax.Precision.HIGHEST,
            preferred_element_type=jnp.float32,
        ) + b_ref[...]

    return pl.pallas_call(
        body,
        out_shape=jax.ShapeDtypeStruct((N, F), jnp.float32),
        grid=(10,),
        in_specs=[
            pl.BlockSpec((NCORES, N // 10, F), lambda i: (0, i, 0)),
            pl.BlockSpec((F, F), lambda i: (0, 0)),
            pl.BlockSpec((1, F), lambda i: (0, 0)),
        ],
        out_specs=pl.BlockSpec((N // 10, F), lambda i: (i, 0)),
    )(p, W, b2)


@jax.jit
def _run(x, edge_index, edge_weight, W, b):
    row3 = edge_index[0].reshape(NTILES, NBLK, CPB, CH)
    col3 = edge_index[1].reshape(NTILES, NBLK, CPB, CH)
    w3 = edge_weight.reshape(NTILES, NBLK, CPB, CH)
    xp = jnp.concatenate([x, jnp.zeros((NPAD - N, F), jnp.float32)], axis=0)
    p1 = _spmm_hop(xp, col3, row3, w3)
    h1 = _combine(p1)
    p2 = _spmm_hop(h1, col3, row3, w3)
    return _final(p2, W, b.reshape(1, F))


def kernel(x, edge_index, edge_weight, W, b):
    return _run(x, edge_index, edge_weight, W, b)


# speedup vs baseline: 1.0598x; 1.0598x over previous
import dataclasses
import functools

import jax
import jax.numpy as jnp
from jax import lax
from jax.experimental import pallas as pl
from jax.experimental.pallas import tpu as pltpu
from jax.experimental.pallas import tpu_sc as plsc

N = 10000
NPAD = 10240
E = 320000
F = 128

NCORES = 2
NSUB = 16
NTILES = NCORES * NSUB
EPT = E // NTILES
CH = 80
NCH = EPT // CH
CPB = 25
NBLK = NCH // CPB
NBUF = 2
ZR = 128
RPT = NPAD // NSUB

_mesh = plsc.VectorSubcoreMesh(core_axis_name="c", subcore_axis_name="s")

_sc_params = pltpu.CompilerParams()
if "needs_layout_passes" in pltpu.CompilerParams.__dataclass_fields__:
    _sc_params = dataclasses.replace(_sc_params, needs_layout_passes=False)


def _spmm_hop(table, col3, row3, w3):
    @functools.partial(
        pl.kernel,
        out_type=jax.ShapeDtypeStruct((NCORES, NPAD, F), jnp.float32),
        mesh=_mesh,
        compiler_params=_sc_params,
        scratch_types=[
            pltpu.VMEM((CPB, CH), jnp.int32),
            pltpu.VMEM((CPB, CH), jnp.int32),
            pltpu.VMEM((CPB, CH), jnp.float32),
            pltpu.VMEM((CH, F), jnp.float32),
            pltpu.VMEM((CH, F), jnp.float32),
            pltpu.VMEM_SHARED((NPAD, F), jnp.float32),
            pltpu.SemaphoreType.DMA,
            pltpu.SemaphoreType.DMA,
        ],
    )
    def hop(table_hbm, col_hbm, row_hbm, w_hbm, out_hbm,
            col_t, row_t, w_t, rows0, rows1, acc, gsem0, gsem1):
        c = lax.axis_index("c")
        s = lax.axis_index("s")
        t = c * NSUB + s
        rows = (rows0, rows1)
        gsem = (gsem0, gsem1)

        @pl.loop(0, RPT // ZR)
        def _(i):
            pltpu.sync_copy(table_hbm.at[pl.ds(N, ZR)],
                            acc.at[pl.ds(s * RPT + i * ZR, ZR)])

        plsc.subcore_barrier()

        def scale(kk, b):
            kv = jnp.full((16,), kk, jnp.int32)

            @pl.loop(0, CH, step=2)
            def _(e):
                for d in range(2):
                    ee = e + d
                    wv = plsc.load_gather(
                        w_t, [kv, jnp.full((16,), ee, jnp.int32)])
                    for j in range(F // 16):
                        sl = (ee, pl.ds(j * 16, 16))
                        rows[b][sl] = rows[b][sl] * wv

        @pl.loop(0, NBLK)
        def _(blk):
            pltpu.sync_copy(col_hbm.at[t].at[blk], col_t)
            pltpu.sync_copy(row_hbm.at[t].at[blk], row_t)
            pltpu.sync_copy(w_hbm.at[t].at[blk], w_t)

            for b in range(NBUF):
                pltpu.async_copy(table_hbm.at[col_t.at[b]], rows[b], gsem[b])

            @pl.loop(0, CPB - 3, step=NBUF)
            def _(k):
                for b in range(NBUF):
                    kk = k + b
                    pltpu.make_async_copy(
                        table_hbm.at[col_t.at[kk]], rows[b], gsem[b]).wait()
                    scale(kk, b)
                    pltpu.sync_copy(rows[b], acc.at[row_t.at[kk]], add=True)
                    pltpu.async_copy(
                        table_hbm.at[col_t.at[kk + NBUF]], rows[b], gsem[b])

            kk = CPB - 3
            pltpu.make_async_copy(table_hbm.at[col_t.at[kk]], rows[0], gsem[0]).wait()
            scale(kk, 0)
            pltpu.sync_copy(rows[0], acc.at[row_t.at[kk]], add=True)
            pltpu.async_copy(table_hbm.at[col_t.at[CPB - 1]], rows[0], gsem[0])
            kk = CPB - 2
            pltpu.make_async_copy(table_hbm.at[col_t.at[kk]], rows[1], gsem[1]).wait()
            scale(kk, 1)
            pltpu.sync_copy(rows[1], acc.at[row_t.at[kk]], add=True)
            kk = CPB - 1
            pltpu.make_async_copy(table_hbm.at[col_t.at[kk]], rows[0], gsem[0]).wait()
            scale(kk, 0)
            pltpu.sync_copy(rows[0], acc.at[row_t.at[kk]], add=True)

        plsc.subcore_barrier()

        @pl.loop(0, RPT // ZR)
        def _(i):
            off = s * RPT + i * ZR
            pltpu.sync_copy(acc.at[pl.ds(off, ZR)], out_hbm.at[c].at[pl.ds(off, ZR)])

    return hop(table, col3, row3, w3)


def _combine(p):
    def body(p_ref, o_ref):
        o_ref[...] = p_ref[0] + p_ref[1]

    return pl.pallas_call(
        body,
        out_shape=jax.ShapeDtypeStruct((NPAD, F), jnp.float32),
        grid=(8,),
        in_specs=[pl.BlockSpec((NCORES, NPAD // 8, F), lambda i: (0, i, 0))],
        out_specs=pl.BlockSpec((NPAD // 8, F), lambda i: (i, 0)),
    )(p)


def _final(p, W, b2):
    def body(p_ref, w_ref, b_ref, o_ref):
        h = p_ref[0] + p_ref[1]
        o_ref[...] = lax.dot_general(
            h, w_ref[...], (((1,), (1,)), ((), ())),
            precision=lax.Precision.HIGHEST,
            preferred_element_type=jnp.float32,
        ) + b_ref[...]

    return pl.pallas_call(
        body,
        out_shape=jax.ShapeDtypeStruct((N, F), jnp.float32),
        grid=(10,),
        in_specs=[
            pl.BlockSpec((NCORES, N // 10, F), lambda i: (0, i, 0)),
            pl.BlockSpec((F, F), lambda i: (0, 0)),
            pl.BlockSpec((1, F), lambda i: (0, 0)),
        ],
        out_specs=pl.BlockSpec((N // 10, F), lambda i: (i, 0)),
    )(p, W, b2)


@jax.jit
def _run(x, edge_index, edge_weight, W, b):
    row3 = edge_index[0].reshape(NTILES, NBLK, CPB, CH)
    col3 = edge_index[1].reshape(NTILES, NBLK, CPB, CH)
    w3 = edge_weight.reshape(NTILES, NBLK, CPB, CH)
    xp = jnp.concatenate([x, jnp.zeros((NPAD - N, F), jnp.float32)], axis=0)
    p1 = _spmm_hop(xp, col3, row3, w3)
    h1 = _combine(p1)
    p2 = _spmm_hop(h1, col3, row3, w3)
    return _final(p2, W, b.reshape(1, F))


def kernel(x, edge_index, edge_weight, W, b):
    return _run(x, edge_index, edge_weight, W, b)
